# Initial kernel scaffold; baseline (speedup 1.0000x reference)
import jax
import jax.numpy as jnp
from jax.experimental import pallas as pl


def kernel(x, edge_index, W1, b1, W2, b2, W3, b3, W4, b4):
    raise NotImplementedError("write your pallas kernel here")


# speedup vs baseline: 24.9098x; 24.9098x over previous
import functools

import jax
import jax.numpy as jnp
from jax import lax
from jax.experimental import pallas as pl
from jax.experimental.pallas import tpu as pltpu
from jax.experimental.pallas import tpu_sc as plsc

NC = 2
NS = 16
NW = NC * NS
CH = 128


def _cdiv(a, b):
    return (a + b - 1) // b


@functools.lru_cache(maxsize=None)
def _make_prop(n, f, rpt, n_acc):
    mesh = plsc.VectorSubcoreMesh(
        core_axis_name="c", subcore_axis_name="s", num_cores=NC, num_subcores=NS)
    zc = n_acc // NS

    @functools.partial(
        pl.kernel, mesh=mesh,
        out_type=jax.ShapeDtypeStruct((NC * n_acc, f), jnp.float32),
        scratch_types=[
            pltpu.VMEM((rpt, CH), jnp.int32),
            pltpu.VMEM((rpt, CH), jnp.int32),
            pltpu.VMEM((CH, f), jnp.float32),
            pltpu.VMEM_SHARED((n_acc, f), jnp.float32),
            pltpu.SemaphoreType.DMA,
        ],
        compiler_params=pltpu.CompilerParams(use_tc_tiling_on_sc=False),
    )
    def prop(y_hbm, src_hbm, dst_hbm, zeros_hbm, out_hbm,
             src_v, dst_v, rows_v, acc, sem):
        c = lax.axis_index("c")
        s = lax.axis_index("s")
        pltpu.sync_copy(zeros_hbm.at[pl.ds(s * zc, zc)], acc.at[pl.ds(s * zc, zc)])
        base = (c * NS + s) * rpt
        pltpu.sync_copy(src_hbm.at[pl.ds(base, rpt)], src_v)
        pltpu.sync_copy(dst_hbm.at[pl.ds(base, rpt)], dst_v)
        plsc.subcore_barrier()

        def body(j, carry):
            pltpu.async_copy(y_hbm.at[src_v.at[j]], rows_v, sem).wait()
            pltpu.sync_copy(rows_v, acc.at[dst_v.at[j]], add=True)
            return carry

        lax.fori_loop(0, rpt, body, 0)
        plsc.subcore_barrier()
        pltpu.sync_copy(acc.at[pl.ds(s * zc, zc)],
                        out_hbm.at[pl.ds(c * n_acc + s * zc, zc)])

    return prop


@functools.lru_cache(maxsize=None)
def _make_deg(n, f, rpt, n_acc):
    mesh = plsc.VectorSubcoreMesh(
        core_axis_name="c", subcore_axis_name="s", num_cores=NC, num_subcores=NS)
    zc = n_acc // NS

    @functools.partial(
        pl.kernel, mesh=mesh,
        out_type=jax.ShapeDtypeStruct((NC * n_acc, f), jnp.float32),
        scratch_types=[
            pltpu.VMEM((rpt, CH), jnp.int32),
            pltpu.VMEM((CH, f), jnp.float32),
            pltpu.VMEM_SHARED((n_acc, f), jnp.float32),
        ],
        compiler_params=pltpu.CompilerParams(use_tc_tiling_on_sc=False),
    )
    def deg(dst_hbm, zeros_hbm, ones_hbm, out_hbm, dst_v, ones_v, acc):
        c = lax.axis_index("c")
        s = lax.axis_index("s")
        pltpu.sync_copy(zeros_hbm.at[pl.ds(s * zc, zc)], acc.at[pl.ds(s * zc, zc)])
        base = (c * NS + s) * rpt
        pltpu.sync_copy(dst_hbm.at[pl.ds(base, rpt)], dst_v)
        pltpu.sync_copy(ones_hbm, ones_v)
        plsc.subcore_barrier()

        def body(j, carry):
            pltpu.sync_copy(ones_v, acc.at[dst_v.at[j]], add=True)
            return carry

        lax.fori_loop(0, rpt, body, 0)
        plsc.subcore_barrier()
        pltpu.sync_copy(acc.at[pl.ds(s * zc, zc)],
                        out_hbm.at[pl.ds(c * n_acc + s * zc, zc)])

    return deg


def _tc1_body(x_ref, w1_ref, degp_ref, y1_ref, dinv_ref, *, n, na):
    d = degp_ref[0:n, 0:1] + degp_ref[na:na + n, 0:1] + 1.0
    dinv = lax.rsqrt(d)
    dinv_ref[...] = dinv
    xw = jnp.dot(x_ref[...], w1_ref[...], preferred_element_type=jnp.float32)
    y1_ref[...] = xw * dinv


def _tc_mid_body(tp_ref, y_ref, dinv_ref, b_ref, w_ref, out_ref, *, n, na):
    dinv = dinv_ref[...]
    t = tp_ref[0:n] + tp_ref[na:na + n] + y_ref[...]
    z = jnp.maximum(dinv * t + b_ref[...], 0.0)
    out_ref[...] = dinv * jnp.dot(z, w_ref[...], preferred_element_type=jnp.float32)


def _tc3_body(tp_ref, y_ref, dinv_ref, b_ref, out_ref, *, n, na):
    dinv = dinv_ref[...]
    t = tp_ref[0:n] + tp_ref[na:na + n] + y_ref[...]
    z = jnp.maximum(dinv * t + b_ref[...], 0.0)
    out_ref[...] = dinv * z


def _tc4_body(tp_ref, y_ref, dinv_ref, w3_ref, b3_ref, w4_ref, b4_ref,
              xhat_ref, zs_ref, *, n, na):
    dinv = dinv_ref[...]
    pz = dinv * (tp_ref[0:n] + tp_ref[na:na + n] + y_ref[...])
    xhat_ref[...] = jnp.dot(pz, w3_ref[...],
                            preferred_element_type=jnp.float32) + b3_ref[...]
    zs_ref[...] = jnp.maximum(
        jnp.dot(pz, w4_ref[...], preferred_element_type=jnp.float32)
        + b4_ref[...], 0.0)


def _ahat_body(a_ref, b_ref, o_ref):
    o_ref[...] = lax.dot_general(
        a_ref[...], b_ref[...], (((1,), (1,)), ((), ())),
        preferred_element_type=jnp.float32)


def kernel(x, edge_index, W1, b1, W2, b2, W3, b3, W4, b4):
    n, d_in = x.shape
    e = edge_index.shape[1]
    f1 = W1.shape[1]
    f2 = W2.shape[1]

    rows = _cdiv(e, CH)
    rpt = _cdiv(rows, NW * 8) * 8
    e_pad = rpt * NW * CH
    n_acc = _cdiv(n + 1, NS * 8) * NS * 8

    src = edge_index[0]
    dst = edge_index[1]
    src_p = jnp.concatenate(
        [src, jnp.zeros((e_pad - e,), jnp.int32)]).reshape(rpt * NW, CH)
    dst_p = jnp.concatenate(
        [dst, jnp.full((e_pad - e,), n, jnp.int32)]).reshape(rpt * NW, CH)

    zeros16 = jnp.zeros((n_acc, f1), jnp.float32)
    zeros8 = jnp.zeros((n_acc, f2), jnp.float32)
    ones8 = jnp.ones((CH, f2), jnp.float32)

    prop16 = _make_prop(n, f1, rpt, n_acc)
    prop8 = _make_prop(n, f2, rpt, n_acc)
    degk = _make_deg(n, f2, rpt, n_acc)

    degp = degk(dst_p, zeros8, ones8)

    y1, dinv = pl.pallas_call(
        functools.partial(_tc1_body, n=n, na=n_acc),
        out_shape=(jax.ShapeDtypeStruct((n, f1), jnp.float32),
                   jax.ShapeDtypeStruct((n, 1), jnp.float32)),
    )(x, W1, degp)

    t1 = prop16(y1, src_p, dst_p, zeros16)
    y2 = pl.pallas_call(
        functools.partial(_tc_mid_body, n=n, na=n_acc),
        out_shape=jax.ShapeDtypeStruct((n, f2), jnp.float32),
    )(t1, y1, dinv, b1.reshape(1, f1), W2)

    t2 = prop8(y2, src_p, dst_p, zeros8)
    y3 = pl.pallas_call(
        functools.partial(_tc3_body, n=n, na=n_acc),
        out_shape=jax.ShapeDtypeStruct((n, f2), jnp.float32),
    )(t2, y2, dinv, b2.reshape(1, f2))

    t3 = prop8(y3, src_p, dst_p, zeros8)
    x_hat, zs = pl.pallas_call(
        functools.partial(_tc4_body, n=n, na=n_acc),
        out_shape=(jax.ShapeDtypeStruct((n, d_in), jnp.float32),
                   jax.ShapeDtypeStruct((n, f2), jnp.float32)),
    )(t3, y3, dinv, W3, b3.reshape(1, d_in), W4, b4.reshape(1, f2))

    bm = 200
    a_hat = pl.pallas_call(
        _ahat_body,
        grid=(n // bm,),
        in_specs=[pl.BlockSpec((bm, f2), lambda i: (i, 0)),
                  pl.BlockSpec((n, f2), lambda i: (0, 0))],
        out_specs=pl.BlockSpec((bm, n), lambda i: (i, 0)),
        out_shape=jax.ShapeDtypeStruct((n, n), jnp.float32),
    )(zs, zs)

    return (x_hat, a_hat)


# speedup vs baseline: 31.2197x; 1.2533x over previous
import functools

import jax
import jax.numpy as jnp
from jax import lax
from jax.experimental import pallas as pl
from jax.experimental.pallas import tpu as pltpu
from jax.experimental.pallas import tpu_sc as plsc

NC = 2
NS = 16
NW = NC * NS
CH = 128


def _cdiv(a, b):
    return (a + b - 1) // b


G = 8


@functools.lru_cache(maxsize=None)
def _make_prop(n, f, rpt, n_acc):
    mesh = plsc.VectorSubcoreMesh(
        core_axis_name="c", subcore_axis_name="s", num_cores=NC, num_subcores=NS)
    zc = n_acc // NS
    ng = rpt // G

    @functools.partial(
        pl.kernel, mesh=mesh,
        out_type=jax.ShapeDtypeStruct((NC * n_acc, f), jnp.float32),
        scratch_types=[
            pltpu.VMEM((rpt, CH), jnp.int32),
            pltpu.VMEM((rpt, CH), jnp.int32),
            pltpu.VMEM((2, G, CH, f), jnp.float32),
            pltpu.VMEM_SHARED((n_acc, f), jnp.float32),
            pltpu.SemaphoreType.DMA,
            pltpu.SemaphoreType.DMA,
        ],
        compiler_params=pltpu.CompilerParams(use_tc_tiling_on_sc=False),
    )
    def prop(y_hbm, src_hbm, dst_hbm, zeros_hbm, out_hbm,
             src_v, dst_v, rows_v, acc, sem0, sem1):
        c = lax.axis_index("c")
        s = lax.axis_index("s")
        sems = (sem0, sem1)
        pltpu.sync_copy(zeros_hbm.at[pl.ds(s * zc, zc)], acc.at[pl.ds(s * zc, zc)])
        base = (c * NS + s) * rpt
        pltpu.sync_copy(src_hbm.at[pl.ds(base, rpt)], src_v)
        pltpu.sync_copy(dst_hbm.at[pl.ds(base, rpt)], dst_v)
        plsc.subcore_barrier()

        def issue(bank, g):
            for k in range(G):
                j = jnp.minimum(g * G + k, rpt - 1)
                pltpu.async_copy(y_hbm.at[src_v.at[j]], rows_v.at[bank, k],
                                 sems[bank])

        def drain(bank):
            for k in range(G):
                pltpu.make_async_copy(y_hbm.at[src_v.at[0]],
                                      rows_v.at[bank, k], sems[bank]).wait()

        def scatter(bank, g):
            for k in range(G):
                j = g * G + k
                pltpu.sync_copy(rows_v.at[bank, k], acc.at[dst_v.at[j]],
                                add=True)

        issue(0, 0)
        issue(1, 1)

        def body(i, carry):
            g0 = 2 * i
            drain(0)
            scatter(0, g0)
            issue(0, g0 + 2)
            drain(1)
            scatter(1, g0 + 1)
            issue(1, g0 + 3)
            return carry

        lax.fori_loop(0, ng // 2 - 1, body, 0)
        drain(0)
        scatter(0, ng - 2)
        drain(1)
        scatter(1, ng - 1)
        plsc.subcore_barrier()
        pltpu.sync_copy(acc.at[pl.ds(s * zc, zc)],
                        out_hbm.at[pl.ds(c * n_acc + s * zc, zc)])

    return prop


@functools.lru_cache(maxsize=None)
def _make_deg(n, f, rpt, n_acc):
    mesh = plsc.VectorSubcoreMesh(
        core_axis_name="c", subcore_axis_name="s", num_cores=NC, num_subcores=NS)
    zc = n_acc // NS

    @functools.partial(
        pl.kernel, mesh=mesh,
        out_type=jax.ShapeDtypeStruct((NC * n_acc, f), jnp.float32),
        scratch_types=[
            pltpu.VMEM((rpt, CH), jnp.int32),
            pltpu.VMEM((CH, f), jnp.float32),
            pltpu.VMEM_SHARED((n_acc, f), jnp.float32),
        ],
        compiler_params=pltpu.CompilerParams(use_tc_tiling_on_sc=False),
    )
    def deg(dst_hbm, zeros_hbm, ones_hbm, out_hbm, dst_v, ones_v, acc):
        c = lax.axis_index("c")
        s = lax.axis_index("s")
        pltpu.sync_copy(zeros_hbm.at[pl.ds(s * zc, zc)], acc.at[pl.ds(s * zc, zc)])
        base = (c * NS + s) * rpt
        pltpu.sync_copy(dst_hbm.at[pl.ds(base, rpt)], dst_v)
        pltpu.sync_copy(ones_hbm, ones_v)
        plsc.subcore_barrier()

        def body(j, carry):
            pltpu.sync_copy(ones_v, acc.at[dst_v.at[j]], add=True)
            return carry

        lax.fori_loop(0, rpt, body, 0)
        plsc.subcore_barrier()
        pltpu.sync_copy(acc.at[pl.ds(s * zc, zc)],
                        out_hbm.at[pl.ds(c * n_acc + s * zc, zc)])

    return deg


def _tc1_body(x_ref, w1_ref, degp_ref, y1_ref, dinv_ref, *, n, na):
    d = degp_ref[0:n, 0:1] + degp_ref[na:na + n, 0:1] + 1.0
    dinv = lax.rsqrt(d)
    dinv_ref[...] = dinv
    xw = jnp.dot(x_ref[...], w1_ref[...], preferred_element_type=jnp.float32)
    y1_ref[...] = xw * dinv


def _tc_mid_body(tp_ref, y_ref, dinv_ref, b_ref, w_ref, out_ref, *, n, na):
    dinv = dinv_ref[...]
    t = tp_ref[0:n] + tp_ref[na:na + n] + y_ref[...]
    z = jnp.maximum(dinv * t + b_ref[...], 0.0)
    out_ref[...] = dinv * jnp.dot(z, w_ref[...], preferred_element_type=jnp.float32)


def _tc3_body(tp_ref, y_ref, dinv_ref, b_ref, out_ref, *, n, na):
    dinv = dinv_ref[...]
    t = tp_ref[0:n] + tp_ref[na:na + n] + y_ref[...]
    z = jnp.maximum(dinv * t + b_ref[...], 0.0)
    out_ref[...] = dinv * z


def _tc4_body(tp_ref, y_ref, dinv_ref, w3_ref, b3_ref, w4_ref, b4_ref,
              xhat_ref, zs_ref, *, n, na):
    dinv = dinv_ref[...]
    pz = dinv * (tp_ref[0:n] + tp_ref[na:na + n] + y_ref[...])
    xhat_ref[...] = jnp.dot(pz, w3_ref[...],
                            preferred_element_type=jnp.float32) + b3_ref[...]
    zs_ref[...] = jnp.maximum(
        jnp.dot(pz, w4_ref[...], preferred_element_type=jnp.float32)
        + b4_ref[...], 0.0)


def _ahat_body(a_ref, b_ref, o_ref):
    o_ref[...] = lax.dot_general(
        a_ref[...], b_ref[...], (((1,), (1,)), ((), ())),
        preferred_element_type=jnp.float32)


def kernel(x, edge_index, W1, b1, W2, b2, W3, b3, W4, b4):
    n, d_in = x.shape
    e = edge_index.shape[1]
    f1 = W1.shape[1]
    f2 = W2.shape[1]

    rows = _cdiv(e, CH)
    rpt = _cdiv(rows, NW * 2 * G) * 2 * G
    e_pad = rpt * NW * CH
    n_acc = _cdiv(n + 1, NS * 8) * NS * 8

    src = edge_index[0]
    dst = edge_index[1]
    src_p = jnp.concatenate(
        [src, jnp.zeros((e_pad - e,), jnp.int32)]).reshape(rpt * NW, CH)
    dst_p = jnp.concatenate(
        [dst, jnp.full((e_pad - e,), n, jnp.int32)]).reshape(rpt * NW, CH)

    zeros16 = jnp.zeros((n_acc, f1), jnp.float32)
    zeros8 = jnp.zeros((n_acc, f2), jnp.float32)
    ones8 = jnp.ones((CH, f2), jnp.float32)

    prop16 = _make_prop(n, f1, rpt, n_acc)
    prop8 = _make_prop(n, f2, rpt, n_acc)
    degk = _make_deg(n, f2, rpt, n_acc)

    degp = degk(dst_p, zeros8, ones8)

    y1, dinv = pl.pallas_call(
        functools.partial(_tc1_body, n=n, na=n_acc),
        out_shape=(jax.ShapeDtypeStruct((n, f1), jnp.float32),
                   jax.ShapeDtypeStruct((n, 1), jnp.float32)),
    )(x, W1, degp)

    t1 = prop16(y1, src_p, dst_p, zeros16)
    y2 = pl.pallas_call(
        functools.partial(_tc_mid_body, n=n, na=n_acc),
        out_shape=jax.ShapeDtypeStruct((n, f2), jnp.float32),
    )(t1, y1, dinv, b1.reshape(1, f1), W2)

    t2 = prop8(y2, src_p, dst_p, zeros8)
    y3 = pl.pallas_call(
        functools.partial(_tc3_body, n=n, na=n_acc),
        out_shape=jax.ShapeDtypeStruct((n, f2), jnp.float32),
    )(t2, y2, dinv, b2.reshape(1, f2))

    t3 = prop8(y3, src_p, dst_p, zeros8)
    x_hat, zs = pl.pallas_call(
        functools.partial(_tc4_body, n=n, na=n_acc),
        out_shape=(jax.ShapeDtypeStruct((n, d_in), jnp.float32),
                   jax.ShapeDtypeStruct((n, f2), jnp.float32)),
    )(t3, y3, dinv, W3, b3.reshape(1, d_in), W4, b4.reshape(1, f2))

    bm = 200
    a_hat = pl.pallas_call(
        _ahat_body,
        grid=(n // bm,),
        in_specs=[pl.BlockSpec((bm, f2), lambda i: (i, 0)),
                  pl.BlockSpec((n, f2), lambda i: (0, 0))],
        out_specs=pl.BlockSpec((bm, n), lambda i: (i, 0)),
        out_shape=jax.ShapeDtypeStruct((n, n), jnp.float32),
    )(zs, zs)

    return (x_hat, a_hat)


# speedup vs baseline: 31.2652x; 1.0015x over previous
import functools

import jax
import jax.numpy as jnp
from jax import lax
from jax.experimental import pallas as pl
from jax.experimental.pallas import tpu as pltpu
from jax.experimental.pallas import tpu_sc as plsc

NC = 2
NS = 16
NW = NC * NS
CH = 128


def _cdiv(a, b):
    return (a + b - 1) // b


G = 8
NB = 4


@functools.lru_cache(maxsize=None)
def _make_prop(n, f, rpt, n_acc):
    mesh = plsc.VectorSubcoreMesh(
        core_axis_name="c", subcore_axis_name="s", num_cores=NC, num_subcores=NS)
    zc = n_acc // NS
    ng = rpt // G

    @functools.partial(
        pl.kernel, mesh=mesh,
        out_type=jax.ShapeDtypeStruct((NC * n_acc, f), jnp.float32),
        scratch_types=[
            pltpu.VMEM((rpt, CH), jnp.int32),
            pltpu.VMEM((rpt, CH), jnp.int32),
            pltpu.VMEM((NB, G, CH, f), jnp.float32),
            pltpu.VMEM_SHARED((n_acc, f), jnp.float32),
            [pltpu.SemaphoreType.DMA] * NB,
            [pltpu.SemaphoreType.DMA] * NB,
            pltpu.SemaphoreType.DMA,
        ],
        compiler_params=pltpu.CompilerParams(use_tc_tiling_on_sc=False),
    )
    def prop(y_hbm, src_hbm, dst_hbm, zeros_hbm, out_hbm,
             src_v, dst_v, rows_v, acc, gsem, ssem, stsem):
        c = lax.axis_index("c")
        s = lax.axis_index("s")
        base = (c * NS + s) * rpt
        pltpu.async_copy(zeros_hbm.at[pl.ds(s * zc, zc)],
                         acc.at[pl.ds(s * zc, zc)], stsem)
        pltpu.async_copy(src_hbm.at[pl.ds(base, rpt)], src_v, stsem)
        pltpu.async_copy(dst_hbm.at[pl.ds(base, rpt)], dst_v, stsem)
        pltpu.make_async_copy(zeros_hbm.at[pl.ds(s * zc, zc)],
                              acc.at[pl.ds(s * zc, zc)], stsem).wait()
        pltpu.make_async_copy(src_hbm.at[pl.ds(base, rpt)], src_v, stsem).wait()
        pltpu.make_async_copy(dst_hbm.at[pl.ds(base, rpt)], dst_v, stsem).wait()
        plsc.subcore_barrier()

        def issue_gather(g):
            b = g % NB
            for k in range(G):
                pltpu.async_copy(y_hbm.at[src_v.at[g * G + k]],
                                 rows_v.at[b, k], gsem[b])

        def drain_gather(g):
            b = g % NB
            for k in range(G):
                pltpu.make_async_copy(y_hbm.at[src_v.at[0]],
                                      rows_v.at[b, k], gsem[b]).wait()

        def issue_scatter(g):
            b = g % NB
            for k in range(G):
                pltpu.async_copy(rows_v.at[b, k], acc.at[dst_v.at[g * G + k]],
                                 ssem[b], add=True)

        def drain_scatter(g):
            b = g % NB
            for k in range(G):
                pltpu.make_async_copy(rows_v.at[b, k],
                                      acc.at[dst_v.at[0]], ssem[b]).wait()

        for g in range(min(NB - 1, ng)):
            issue_gather(g)
        for g in range(ng):
            drain_gather(g)
            issue_scatter(g)
            h = g + NB - 1
            if h < ng:
                if h >= NB:
                    drain_scatter(h - NB)
                issue_gather(h)
        for g in range(max(0, ng - NB), ng):
            drain_scatter(g)
        plsc.subcore_barrier()
        pltpu.sync_copy(acc.at[pl.ds(s * zc, zc)],
                        out_hbm.at[pl.ds(c * n_acc + s * zc, zc)])

    return prop


@functools.lru_cache(maxsize=None)
def _make_deg(n, f, rpt, n_acc):
    mesh = plsc.VectorSubcoreMesh(
        core_axis_name="c", subcore_axis_name="s", num_cores=NC, num_subcores=NS)
    zc = n_acc // NS

    @functools.partial(
        pl.kernel, mesh=mesh,
        out_type=jax.ShapeDtypeStruct((NC * n_acc, f), jnp.float32),
        scratch_types=[
            pltpu.VMEM((rpt, CH), jnp.int32),
            pltpu.VMEM((CH, f), jnp.float32),
            pltpu.VMEM_SHARED((n_acc, f), jnp.float32),
            pltpu.SemaphoreType.DMA,
        ],
        compiler_params=pltpu.CompilerParams(use_tc_tiling_on_sc=False),
    )
    def deg(dst_hbm, zeros_hbm, ones_hbm, out_hbm, dst_v, ones_v, acc, sem):
        c = lax.axis_index("c")
        s = lax.axis_index("s")
        pltpu.sync_copy(zeros_hbm.at[pl.ds(s * zc, zc)], acc.at[pl.ds(s * zc, zc)])
        base = (c * NS + s) * rpt
        pltpu.sync_copy(dst_hbm.at[pl.ds(base, rpt)], dst_v)
        pltpu.sync_copy(ones_hbm, ones_v)
        plsc.subcore_barrier()

        def body(j, carry):
            pltpu.async_copy(ones_v, acc.at[dst_v.at[j]], sem, add=True)
            return carry

        lax.fori_loop(0, rpt, body, 0)

        def dbody(j, carry):
            pltpu.make_async_copy(ones_v, acc.at[dst_v.at[0]], sem).wait()
            return carry

        lax.fori_loop(0, rpt, dbody, 0)
        plsc.subcore_barrier()
        pltpu.sync_copy(acc.at[pl.ds(s * zc, zc)],
                        out_hbm.at[pl.ds(c * n_acc + s * zc, zc)])

    return deg


def _tc1_body(x_ref, w1_ref, degp_ref, y1_ref, dinv_ref, *, n, na):
    d = degp_ref[0:n, 0:1] + degp_ref[na:na + n, 0:1] + 1.0
    dinv = lax.rsqrt(d)
    dinv_ref[...] = dinv
    xw = jnp.dot(x_ref[...], w1_ref[...], preferred_element_type=jnp.float32)
    y1_ref[...] = xw * dinv


def _tc_mid_body(tp_ref, y_ref, dinv_ref, b_ref, w_ref, out_ref, *, n, na):
    dinv = dinv_ref[...]
    t = tp_ref[0:n] + tp_ref[na:na + n] + y_ref[...]
    z = jnp.maximum(dinv * t + b_ref[...], 0.0)
    out_ref[...] = dinv * jnp.dot(z, w_ref[...], preferred_element_type=jnp.float32)


def _tc3_body(tp_ref, y_ref, dinv_ref, b_ref, out_ref, *, n, na):
    dinv = dinv_ref[...]
    t = tp_ref[0:n] + tp_ref[na:na + n] + y_ref[...]
    z = jnp.maximum(dinv * t + b_ref[...], 0.0)
    out_ref[...] = dinv * z


def _tc4_body(tp_ref, y_ref, dinv_ref, w3_ref, b3_ref, w4_ref, b4_ref,
              xhat_ref, zs_ref, *, n, na):
    dinv = dinv_ref[...]
    pz = dinv * (tp_ref[0:n] + tp_ref[na:na + n] + y_ref[...])
    xhat_ref[...] = jnp.dot(pz, w3_ref[...],
                            preferred_element_type=jnp.float32) + b3_ref[...]
    zs_ref[...] = jnp.maximum(
        jnp.dot(pz, w4_ref[...], preferred_element_type=jnp.float32)
        + b4_ref[...], 0.0)


def _ahat_body(a_ref, b_ref, o_ref):
    o_ref[...] = lax.dot_general(
        a_ref[...], b_ref[...], (((1,), (1,)), ((), ())),
        preferred_element_type=jnp.float32)


def kernel(x, edge_index, W1, b1, W2, b2, W3, b3, W4, b4):
    n, d_in = x.shape
    e = edge_index.shape[1]
    f1 = W1.shape[1]
    f2 = W2.shape[1]

    rows = _cdiv(e, CH)
    rpt = _cdiv(rows, NW * 2 * G) * 2 * G
    e_pad = rpt * NW * CH
    n_acc = _cdiv(n + 1, NS * 8) * NS * 8

    src = edge_index[0]
    dst = edge_index[1]
    src_p = jnp.concatenate(
        [src, jnp.zeros((e_pad - e,), jnp.int32)]).reshape(rpt * NW, CH)
    dst_p = jnp.concatenate(
        [dst, jnp.full((e_pad - e,), n, jnp.int32)]).reshape(rpt * NW, CH)

    zeros16 = jnp.zeros((n_acc, f1), jnp.float32)
    zeros8 = jnp.zeros((n_acc, f2), jnp.float32)
    ones8 = jnp.ones((CH, f2), jnp.float32)

    prop16 = _make_prop(n, f1, rpt, n_acc)
    prop8 = _make_prop(n, f2, rpt, n_acc)
    degk = _make_deg(n, f2, rpt, n_acc)

    degp = degk(dst_p, zeros8, ones8)

    y1, dinv = pl.pallas_call(
        functools.partial(_tc1_body, n=n, na=n_acc),
        out_shape=(jax.ShapeDtypeStruct((n, f1), jnp.float32),
                   jax.ShapeDtypeStruct((n, 1), jnp.float32)),
    )(x, W1, degp)

    t1 = prop16(y1, src_p, dst_p, zeros16)
    y2 = pl.pallas_call(
        functools.partial(_tc_mid_body, n=n, na=n_acc),
        out_shape=jax.ShapeDtypeStruct((n, f2), jnp.float32),
    )(t1, y1, dinv, b1.reshape(1, f1), W2)

    t2 = prop8(y2, src_p, dst_p, zeros8)
    y3 = pl.pallas_call(
        functools.partial(_tc3_body, n=n, na=n_acc),
        out_shape=jax.ShapeDtypeStruct((n, f2), jnp.float32),
    )(t2, y2, dinv, b2.reshape(1, f2))

    t3 = prop8(y3, src_p, dst_p, zeros8)
    x_hat, zs = pl.pallas_call(
        functools.partial(_tc4_body, n=n, na=n_acc),
        out_shape=(jax.ShapeDtypeStruct((n, d_in), jnp.float32),
                   jax.ShapeDtypeStruct((n, f2), jnp.float32)),
    )(t3, y3, dinv, W3, b3.reshape(1, d_in), W4, b4.reshape(1, f2))

    bm = 200
    a_hat = pl.pallas_call(
        _ahat_body,
        grid=(n // bm,),
        in_specs=[pl.BlockSpec((bm, f2), lambda i: (i, 0)),
                  pl.BlockSpec((n, f2), lambda i: (0, 0))],
        out_specs=pl.BlockSpec((bm, n), lambda i: (i, 0)),
        out_shape=jax.ShapeDtypeStruct((n, n), jnp.float32),
    )(zs, zs)

    return (x_hat, a_hat)
